# baseline (speedup 1.0000x reference)
import functools

import jax
import jax.numpy as jnp
from jax.experimental import pallas as pl


def _round_to_bf16_f32(v):
    u = jax.lax.bitcast_convert_type(v, jnp.uint32)
    lsb = jax.lax.shift_right_logical(u, jnp.uint32(16)) & jnp.uint32(1)
    u = u + jnp.uint32(0x7FFF) + lsb
    u = u & jnp.uint32(0xFFFF0000)
    return jax.lax.bitcast_convert_type(u, jnp.float32)


def _rq_tile_kernel(x_ref, cb0_ref, cb1_ref, cb2_ref, cb3_ref,
                    codes_ref, msesum_ref, *, n_stages):
    x = x_ref[...]
    tb = x.shape[0]
    cb_refs = (cb0_ref, cb1_ref, cb2_ref, cb3_ref)[:n_stages]
    xq = jnp.zeros_like(x)
    for s, cb_ref in enumerate(cb_refs):
        cb = cb_ref[...]
        k = cb.shape[0]
        r = x - xq
        c2 = jnp.sum(cb * cb, axis=1)
        x2 = jnp.sum(r * r, axis=1, keepdims=True)
        xc = jax.lax.dot_general(_round_to_bf16_f32(r), _round_to_bf16_f32(cb),
                                 (((1,), (1,)), ((), ())),
                                 preferred_element_type=jnp.float32)
        dists = x2 - 2.0 * xc + c2[None, :]
        c = jnp.argmin(dists, axis=1).astype(jnp.int32)
        codes_ref[0, s, :] = c
        onehot = (jax.lax.broadcasted_iota(jnp.int32, (tb, k), 1)
                  == c[:, None]).astype(jnp.float32)
        sel = jax.lax.dot_general(onehot, cb, (((1,), (0,)), ((), ())),
                                  precision=jax.lax.Precision.HIGHEST,
                                  preferred_element_type=jnp.float32)
        xq = xq + sel

    @pl.when(pl.program_id(0) == 0)
    def _init():
        msesum_ref[...] = jnp.zeros((1, 1), jnp.float32)

    r_fin = x - xq
    msesum_ref[...] += jnp.sum(r_fin * r_fin)[None, None]


def kernel(x_in, cb0, cb1, cb2, cb3):
    b, d = x_in.shape
    k = cb0.shape[0]
    m = 4
    tb = min(256, b)
    nb = b // tb

    codes3, msesum = pl.pallas_call(
        functools.partial(_rq_tile_kernel, n_stages=m),
        grid=(nb,),
        in_specs=[
            pl.BlockSpec((tb, d), lambda i: (i, 0)),
            pl.BlockSpec((k, d), lambda i: (0, 0)),
            pl.BlockSpec((k, d), lambda i: (0, 0)),
            pl.BlockSpec((k, d), lambda i: (0, 0)),
            pl.BlockSpec((k, d), lambda i: (0, 0)),
        ],
        out_specs=[
            pl.BlockSpec((1, m, tb), lambda i: (i, 0, 0)),
            pl.BlockSpec((1, 1), lambda i: (0, 0)),
        ],
        out_shape=[
            jax.ShapeDtypeStruct((nb, m, tb), jnp.int32),
            jax.ShapeDtypeStruct((1, 1), jnp.float32),
        ],
    )(x_in, cb0, cb1, cb2, cb3)

    codes = codes3.transpose(1, 0, 2).reshape(m, b)
    mse = msesum[0, 0] / jnp.float32(b * d)
    return codes, mse
